# Initial kernel scaffold; baseline (speedup 1.0000x reference)
import jax
import jax.numpy as jnp
from jax.experimental import pallas as pl


def kernel(node_ids, edge_index, W_emb, b_emb, W1, b1, W2, b2):
    raise NotImplementedError("write your pallas kernel here")


# speedup vs baseline: 2.4280x; 2.4280x over previous
import functools

import jax
import jax.numpy as jnp
from jax import lax
from jax.experimental import pallas as pl
from jax.experimental.pallas import tpu as pltpu

NUM_NODES = 10000
EMBED = 256
E = 160000

M_BLK = 400


def _k1_body(nid_ref, wemb_ref, bemb_ref, w1_ref, dinv_ref, h0_ref, y_ref):
    h0 = jnp.dot(nid_ref[...], wemb_ref[...],
                 preferred_element_type=jnp.float32) + bemb_ref[...]
    h0_ref[...] = h0
    x1 = jnp.dot(h0, w1_ref[...], preferred_element_type=jnp.float32)
    y_ref[...] = x1 * dinv_ref[...]


def _k1(node_ids, w_emb, b_emb2d, w1, dinv2d):
    grid = (NUM_NODES // M_BLK,)
    return pl.pallas_call(
        _k1_body,
        grid=grid,
        in_specs=[
            pl.BlockSpec((M_BLK, NUM_NODES), lambda i: (i, 0)),
            pl.BlockSpec((NUM_NODES, EMBED), lambda i: (0, 0)),
            pl.BlockSpec((1, EMBED), lambda i: (0, 0)),
            pl.BlockSpec((EMBED, EMBED), lambda i: (0, 0)),
            pl.BlockSpec((M_BLK, 1), lambda i: (i, 0)),
        ],
        out_specs=[
            pl.BlockSpec((M_BLK, EMBED), lambda i: (i, 0)),
            pl.BlockSpec((M_BLK, EMBED), lambda i: (i, 0)),
        ],
        out_shape=[
            jax.ShapeDtypeStruct((NUM_NODES, EMBED), jnp.float32),
            jax.ShapeDtypeStruct((NUM_NODES, EMBED), jnp.float32),
        ],
    )(node_ids, w_emb, b_emb2d, w1, dinv2d)


def _k3_body(agg_ref, y_ref, dinv_ref, b1_ref, w2_ref, y2_ref):
    h1 = jax.nn.relu((agg_ref[...] + y_ref[...]) * dinv_ref[...] + b1_ref[...])
    x2 = jnp.dot(h1, w2_ref[...], preferred_element_type=jnp.float32)
    y2_ref[...] = x2 * dinv_ref[...]


def _k3(agg1, y1, dinv2d, b1_2d, w2):
    grid = (NUM_NODES // 1000,)
    return pl.pallas_call(
        _k3_body,
        grid=grid,
        in_specs=[
            pl.BlockSpec((1000, EMBED), lambda i: (i, 0)),
            pl.BlockSpec((1000, EMBED), lambda i: (i, 0)),
            pl.BlockSpec((1000, 1), lambda i: (i, 0)),
            pl.BlockSpec((1, EMBED), lambda i: (0, 0)),
            pl.BlockSpec((EMBED, EMBED), lambda i: (0, 0)),
        ],
        out_specs=pl.BlockSpec((1000, EMBED), lambda i: (i, 0)),
        out_shape=jax.ShapeDtypeStruct((NUM_NODES, EMBED), jnp.float32),
    )(agg1, y1, dinv2d, b1_2d, w2)


def _k5_body(agg_ref, y2_ref, dinv_ref, b2_ref, h0_ref, out_ref):
    out_ref[...] = ((agg_ref[...] + y2_ref[...]) * dinv_ref[...]
                    + b2_ref[...] + h0_ref[...])


def _k5(agg2, y2, dinv2d, b2_2d, h0):
    grid = (NUM_NODES // 1000,)
    return pl.pallas_call(
        _k5_body,
        grid=grid,
        in_specs=[
            pl.BlockSpec((1000, EMBED), lambda i: (i, 0)),
            pl.BlockSpec((1000, EMBED), lambda i: (i, 0)),
            pl.BlockSpec((1000, 1), lambda i: (i, 0)),
            pl.BlockSpec((1, EMBED), lambda i: (0, 0)),
            pl.BlockSpec((1000, EMBED), lambda i: (i, 0)),
        ],
        out_specs=pl.BlockSpec((1000, EMBED), lambda i: (i, 0)),
        out_shape=jax.ShapeDtypeStruct((NUM_NODES, EMBED), jnp.float32),
    )(agg2, y2, dinv2d, b2_2d, h0)


def kernel(node_ids, edge_index, W_emb, b_emb, W1, b1, W2, b2):
    src = edge_index[0]
    dst = edge_index[1]

    deg = jnp.ones((NUM_NODES,), jnp.float32).at[dst].add(
        1.0, mode="drop", indices_are_sorted=False, unique_indices=False)
    dinv2d = lax.rsqrt(deg)[:, None]

    h0, y1 = _k1(node_ids, W_emb, b_emb[None, :], W1, dinv2d)

    agg1 = jnp.zeros((NUM_NODES, EMBED), jnp.float32).at[dst].add(y1[src])
    y2 = _k3(agg1, y1, dinv2d, b1[None, :], W2)
    agg2 = jnp.zeros((NUM_NODES, EMBED), jnp.float32).at[dst].add(y2[src])
    return _k5(agg2, y2, dinv2d, b2[None, :], h0)


# speedup vs baseline: 7.8155x; 3.2189x over previous
import functools

import jax
import jax.numpy as jnp
from jax import lax
from jax.experimental import pallas as pl
from jax.experimental.pallas import tpu as pltpu
from jax.experimental.pallas import tpu_sc as plsc

NUM_NODES = 10000
EMBED = 256
HALF = 128
E = 160000

M_BLK = 400
N_SUB = 16
EDGES_PER_TILE = E // N_SUB
BATCH = 125
N_BATCH = EDGES_PER_TILE // BATCH
SLAB = 624
ZCH = 104


def _k1_body(nid_ref, wemb_ref, bemb_ref, w1_ref, dinv_ref,
             h0_ref, y0_ref, y1_ref):
    h0 = jnp.dot(nid_ref[...], wemb_ref[...],
                 preferred_element_type=jnp.float32) + bemb_ref[...]
    h0_ref[...] = h0
    y = jnp.dot(h0, w1_ref[...], preferred_element_type=jnp.float32) * dinv_ref[...]
    y0_ref[...] = y[:, :HALF]
    y1_ref[...] = y[:, HALF:]


def _k1(node_ids, w_emb, b_emb2d, w1, dinv2d):
    grid = (NUM_NODES // M_BLK,)
    return pl.pallas_call(
        _k1_body,
        grid=grid,
        in_specs=[
            pl.BlockSpec((M_BLK, NUM_NODES), lambda i: (i, 0)),
            pl.BlockSpec((NUM_NODES, EMBED), lambda i: (0, 0)),
            pl.BlockSpec((1, EMBED), lambda i: (0, 0)),
            pl.BlockSpec((EMBED, EMBED), lambda i: (0, 0)),
            pl.BlockSpec((M_BLK, 1), lambda i: (i, 0)),
        ],
        out_specs=[
            pl.BlockSpec((M_BLK, EMBED), lambda i: (i, 0)),
            pl.BlockSpec((M_BLK, HALF), lambda i: (i, 0)),
            pl.BlockSpec((M_BLK, HALF), lambda i: (i, 0)),
        ],
        out_shape=[
            jax.ShapeDtypeStruct((NUM_NODES, EMBED), jnp.float32),
            jax.ShapeDtypeStruct((NUM_NODES, HALF), jnp.float32),
            jax.ShapeDtypeStruct((NUM_NODES, HALF), jnp.float32),
        ],
    )(node_ids, w_emb, b_emb2d, w1, dinv2d)


def _k3_body(a0_ref, a1_ref, y0_ref, y1_ref, dinv_ref, b1_ref, w2_ref,
             y2a_ref, y2b_ref):
    h1a = (a0_ref[...] + y0_ref[...]) * dinv_ref[...]
    h1b = (a1_ref[...] + y1_ref[...]) * dinv_ref[...]
    h1 = jax.nn.relu(jnp.concatenate([h1a, h1b], axis=1) + b1_ref[...])
    y2 = jnp.dot(h1, w2_ref[...], preferred_element_type=jnp.float32) * dinv_ref[...]
    y2a_ref[...] = y2[:, :HALF]
    y2b_ref[...] = y2[:, HALF:]


def _k3(a0, a1, y0, y1, dinv2d, b1_2d, w2):
    blk = 1000
    grid = (NUM_NODES // blk,)
    half_in = pl.BlockSpec((blk, HALF), lambda i: (i, 0))
    return pl.pallas_call(
        _k3_body,
        grid=grid,
        in_specs=[
            half_in, half_in, half_in, half_in,
            pl.BlockSpec((blk, 1), lambda i: (i, 0)),
            pl.BlockSpec((1, EMBED), lambda i: (0, 0)),
            pl.BlockSpec((EMBED, EMBED), lambda i: (0, 0)),
        ],
        out_specs=[half_in, half_in],
        out_shape=[
            jax.ShapeDtypeStruct((NUM_NODES, HALF), jnp.float32),
            jax.ShapeDtypeStruct((NUM_NODES, HALF), jnp.float32),
        ],
    )(a0, a1, y0, y1, dinv2d, b1_2d, w2)


def _k5_body(a0_ref, a1_ref, y0_ref, y1_ref, dinv_ref, b2_ref, h0_ref,
             out_ref):
    oa = (a0_ref[...] + y0_ref[...]) * dinv_ref[...]
    ob = (a1_ref[...] + y1_ref[...]) * dinv_ref[...]
    out_ref[...] = jnp.concatenate([oa, ob], axis=1) + b2_ref[...] + h0_ref[...]


def _k5(a0, a1, y0, y1, dinv2d, b2_2d, h0):
    blk = 1000
    grid = (NUM_NODES // blk,)
    half_in = pl.BlockSpec((blk, HALF), lambda i: (i, 0))
    return pl.pallas_call(
        _k5_body,
        grid=grid,
        in_specs=[
            half_in, half_in, half_in, half_in,
            pl.BlockSpec((blk, 1), lambda i: (i, 0)),
            pl.BlockSpec((1, EMBED), lambda i: (0, 0)),
            pl.BlockSpec((blk, EMBED), lambda i: (i, 0)),
        ],
        out_specs=pl.BlockSpec((blk, EMBED), lambda i: (i, 0)),
        out_shape=jax.ShapeDtypeStruct((NUM_NODES, EMBED), jnp.float32),
    )(a0, a1, y0, y1, dinv2d, b2_2d, h0)


def _sc_agg_body(y0_hbm, y1_hbm, src_hbm, dst_hbm, out0_hbm, out1_hbm,
                 src_v, dst_v, rows_v, agg_sh):
    c = lax.axis_index("c")
    s = lax.axis_index("s")
    base = s * SLAB

    def _zero_row(i, carry):
        for j in range(HALF // 16):
            rows_v[i, pl.ds(j * 16, 16)] = jnp.zeros((16,), jnp.float32)
        return carry
    lax.fori_loop(0, ZCH, _zero_row, 0)
    for i in range(SLAB // ZCH):
        pltpu.sync_copy(rows_v.at[pl.ds(0, ZCH), :],
                        agg_sh.at[pl.ds(base + i * ZCH, ZCH), :])

    @pl.when(s == N_SUB - 1)
    def _zero_tail():
        pltpu.sync_copy(rows_v.at[pl.ds(0, 16), :],
                        agg_sh.at[pl.ds(N_SUB * SLAB, 16), :])

    pltpu.sync_copy(src_hbm.at[s], src_v)
    pltpu.sync_copy(dst_hbm.at[s], dst_v)
    plsc.subcore_barrier()

    def _run(y_ref):
        def body(b, carry):
            pltpu.sync_copy(y_ref.at[src_v.at[b]], rows_v.at[pl.ds(0, BATCH), :])
            pltpu.sync_copy(rows_v.at[pl.ds(0, BATCH), :],
                            agg_sh.at[dst_v.at[b]], add=True)
            return carry
        lax.fori_loop(0, N_BATCH, body, 0)

    pl.when(c == 0)(lambda: _run(y0_hbm))
    pl.when(c == 1)(lambda: _run(y1_hbm))
    plsc.subcore_barrier()

    def _writeback(out_ref):
        pltpu.sync_copy(agg_sh.at[pl.ds(base, SLAB), :],
                        out_ref.at[pl.ds(base, SLAB), :])

        @pl.when(s == N_SUB - 1)
        def _tail():
            pltpu.sync_copy(agg_sh.at[pl.ds(N_SUB * SLAB, 16), :],
                            out_ref.at[pl.ds(N_SUB * SLAB, 16), :])
    pl.when(c == 0)(lambda: _writeback(out0_hbm))
    pl.when(c == 1)(lambda: _writeback(out1_hbm))


def _sc_agg(y0, y1, src3, dst3):
    mesh = plsc.VectorSubcoreMesh(core_axis_name="c", subcore_axis_name="s")
    return pl.kernel(
        _sc_agg_body,
        out_type=[
            jax.ShapeDtypeStruct((NUM_NODES, HALF), jnp.float32),
            jax.ShapeDtypeStruct((NUM_NODES, HALF), jnp.float32),
        ],
        mesh=mesh,
        scratch_types=[
            pltpu.VMEM((N_BATCH, BATCH), jnp.int32),
            pltpu.VMEM((N_BATCH, BATCH), jnp.int32),
            pltpu.VMEM((HALF, HALF), jnp.float32),
            pltpu.VMEM_SHARED((NUM_NODES, HALF), jnp.float32),
        ],
    )(y0, y1, src3, dst3)


def kernel(node_ids, edge_index, W_emb, b_emb, W1, b1, W2, b2):
    src = edge_index[0]
    dst = edge_index[1]
    src3 = src.reshape(N_SUB, N_BATCH, BATCH)
    dst3 = dst.reshape(N_SUB, N_BATCH, BATCH)

    deg = jnp.ones((NUM_NODES,), jnp.float32).at[dst].add(1.0)
    dinv2d = lax.rsqrt(deg)[:, None]

    h0, y1_0, y1_1 = _k1(node_ids, W_emb, b_emb[None, :], W1, dinv2d)
    a1_0, a1_1 = _sc_agg(y1_0, y1_1, src3, dst3)
    y2_0, y2_1 = _k3(a1_0, a1_1, y1_0, y1_1, dinv2d, b1[None, :], W2)
    a2_0, a2_1 = _sc_agg(y2_0, y2_1, src3, dst3)
    return _k5(a2_0, a2_1, y2_0, y2_1, dinv2d, b2[None, :], h0)
